# Initial kernel scaffold; baseline (speedup 1.0000x reference)
import functools

import jax
import jax.numpy as jnp
from jax import lax
from jax.experimental import pallas as pl
from jax.experimental.pallas import tpu as pltpu
from jax.experimental.pallas import tpu_sc as plsc

NC = 2
NS = 16
NW = NC * NS
CHUNK = 80
WB = 10

_mesh = plsc.VectorSubcoreMesh(core_axis_name="c", subcore_axis_name="s")


def _deg_body(n, nch, dst_hbm, zero_hbm, out_hbm, idx_v, ones_v, acc_sh, sem):
    c = lax.axis_index("c")
    s = lax.axis_index("s")
    wid = s * NC + c
    rows = n // WB

    pltpu.sync_copy(dst_hbm.at[pl.ds(wid * nch, nch)], idx_v)
    for i in range(CHUNK // 16):
        ones_v[pl.ds(i * 16, 16)] = jnp.ones((16,), jnp.float32)

    @pl.when(s < WB)
    def _():
        pltpu.sync_copy(zero_hbm.at[pl.ds(s * rows, rows)],
                        acc_sh.at[pl.ds(s * rows, rows)])

    plsc.subcore_barrier()

    def body(j, carry):
        pltpu.sync_copy(ones_v, acc_sh.at[idx_v.at[j]], add=True)
        return carry

    lax.fori_loop(0, nch, body, 0)
    plsc.subcore_barrier()

    @pl.when(s < WB)
    def _():
        pltpu.sync_copy(acc_sh.at[pl.ds(s * rows, rows)],
                        out_hbm.at[c, pl.ds(s * rows, rows)])


def _agg_body(n, d, nch, src_hbm, dst_hbm, h_hbm, zero_hbm, out_hbm,
              srcv, dstv, rows_v, acc_sh, sem):
    c = lax.axis_index("c")
    s = lax.axis_index("s")
    wid = s * NC + c
    rows = n // WB

    pltpu.sync_copy(src_hbm.at[pl.ds(wid * nch, nch)], srcv)
    pltpu.sync_copy(dst_hbm.at[pl.ds(wid * nch, nch)], dstv)

    @pl.when(s < WB)
    def _():
        pltpu.sync_copy(zero_hbm.at[pl.ds(s * rows, rows)],
                        acc_sh.at[pl.ds(s * rows, rows)])

    plsc.subcore_barrier()

    def body(j, carry):
        pltpu.async_copy(h_hbm.at[srcv.at[j]], rows_v, sem).wait()
        pltpu.sync_copy(rows_v, acc_sh.at[dstv.at[j]], add=True)
        return carry

    lax.fori_loop(0, nch, body, 0)
    plsc.subcore_barrier()

    @pl.when(s < WB)
    def _():
        pltpu.sync_copy(acc_sh.at[pl.ds(s * rows, rows)],
                        out_hbm.at[c, pl.ds(s * rows, rows)])


def _make_deg(n, e):
    nch = e // (NW * CHUNK)
    return functools.partial(
        pl.kernel,
        mesh=_mesh,
        out_type=jax.ShapeDtypeStruct((NC, n), jnp.float32),
        scratch_types=[
            pltpu.VMEM((nch, CHUNK), jnp.int32),
            pltpu.VMEM((CHUNK,), jnp.float32),
            pltpu.VMEM_SHARED((n,), jnp.float32),
            pltpu.SemaphoreType.DMA,
        ],
    )(functools.partial(_deg_body, n, nch))


def _make_agg(n, d, e):
    nch = e // (NW * CHUNK)
    return functools.partial(
        pl.kernel,
        mesh=_mesh,
        out_type=jax.ShapeDtypeStruct((NC, n, d), jnp.float32),
        scratch_types=[
            pltpu.VMEM((nch, CHUNK), jnp.int32),
            pltpu.VMEM((nch, CHUNK), jnp.int32),
            pltpu.VMEM((CHUNK, d), jnp.float32),
            pltpu.VMEM_SHARED((n, d), jnp.float32),
            pltpu.SemaphoreType.DMA,
        ],
    )(functools.partial(_agg_body, n, d, nch))


def _mm_body(x_ref, w_ref, o_ref):
    o_ref[...] = jnp.dot(x_ref[...], w_ref[...],
                         preferred_element_type=jnp.float32)


def _scale_body(indeg_ref, g_ref, dis_ref, hp_ref):
    deg = indeg_ref[0] + indeg_ref[1] + 1.0
    dis = lax.rsqrt(deg)
    dis_ref[...] = dis
    hp_ref[...] = dis[:, None] * g_ref[...]


def _mid_body(dis_ref, agg_ref, hp_ref, b1_ref, w2_ref, h2p_ref):
    dis = dis_ref[...]
    a = agg_ref[0] + agg_ref[1] + hp_ref[...]
    h1 = jnp.maximum(dis[:, None] * a + b1_ref[...][None, :], 0.0)
    g2 = jnp.dot(h1, w2_ref[...], preferred_element_type=jnp.float32)
    h2p_ref[...] = dis[:, None] * g2


def _out_body(dis_ref, agg_ref, h2p_ref, b2_ref, o_ref):
    dis = dis_ref[...]
    a = agg_ref[0] + agg_ref[1] + h2p_ref[...]
    o_ref[...] = dis[:, None] * a + b2_ref[...][None, :]


def kernel(x, edge_index, W1, b1, W2, b2):
    n, _ = x.shape
    e = edge_index.shape[1]
    hid = W1.shape[1]
    out_dim = W2.shape[1]
    assert e % (NW * CHUNK) == 0 and n % (WB * 8) == 0

    ei = edge_index.astype(jnp.int32)
    src = ei[0].reshape(e // CHUNK, CHUNK)
    dst = ei[1].reshape(e // CHUNK, CHUNK)
    zeros1 = jnp.zeros((n,), jnp.float32)
    zeros_h = jnp.zeros((n, hid), jnp.float32)
    zeros_o = jnp.zeros((n, out_dim), jnp.float32)

    g1 = pl.pallas_call(
        _mm_body,
        out_shape=jax.ShapeDtypeStruct((n, hid), jnp.float32),
    )(x, W1)

    indeg = _make_deg(n, e)(dst, zeros1)

    dis, h1p = pl.pallas_call(
        _scale_body,
        out_shape=(jax.ShapeDtypeStruct((n,), jnp.float32),
                   jax.ShapeDtypeStruct((n, hid), jnp.float32)),
    )(indeg, g1)

    agg1 = _make_agg(n, hid, e)(src, dst, h1p, zeros_h)

    h2p = pl.pallas_call(
        _mid_body,
        out_shape=jax.ShapeDtypeStruct((n, out_dim), jnp.float32),
    )(dis, agg1, h1p, b1, W2)

    agg2 = _make_agg(n, out_dim, e)(src, dst, h2p, zeros_o)

    out = pl.pallas_call(
        _out_body,
        out_shape=jax.ShapeDtypeStruct((n, out_dim), jnp.float32),
    )(dis, agg2, h2p, b2)

    return out


# speedup vs baseline: 12.2662x; 12.2662x over previous
import functools

import jax
import jax.numpy as jnp
from jax import lax
from jax.experimental import pallas as pl
from jax.experimental.pallas import tpu as pltpu
from jax.experimental.pallas import tpu_sc as plsc

NC = 2
NS = 16
NW = NC * NS
CHUNK = 80


def _mesh():
    return plsc.VectorSubcoreMesh(core_axis_name="c", subcore_axis_name="s")


def _deg_body(n, nch, dst_hbm, zero_hbm, out_hbm, idx_v, ones_v, acc_sh, sem):
    c = lax.axis_index("c")
    s = lax.axis_index("s")
    wid = s * NC + c

    pltpu.sync_copy(dst_hbm.at[wid], idx_v)
    for i in range(CHUNK // 16):
        ones_v[pl.ds(i * 16, 16)] = jnp.ones((16,), jnp.float32)

    @pl.when(s == 0)
    def _():
        pltpu.sync_copy(zero_hbm, acc_sh)

    plsc.subcore_barrier()

    def body(j, carry):
        pltpu.sync_copy(ones_v, acc_sh.at[idx_v.at[j]], add=True)
        return carry

    lax.fori_loop(0, nch, body, 0)
    plsc.subcore_barrier()

    @pl.when(s == 0)
    def _():
        pltpu.sync_copy(acc_sh, out_hbm.at[c])


def _agg_body(nch, goffs_hbm, soffs_hbm, h_hbm, zero_hbm, out_hbm,
              gv, sv, rows_v, h_sh, acc_sh, sem):
    c = lax.axis_index("c")
    s = lax.axis_index("s")
    wid = s * NC + c

    @pl.when(s == 0)
    def _():
        pltpu.sync_copy(zero_hbm, acc_sh)
        pltpu.sync_copy(h_hbm, h_sh)

    plsc.subcore_barrier()

    def body(j, carry):
        pltpu.sync_copy(goffs_hbm.at[wid, j], gv)
        pltpu.sync_copy(soffs_hbm.at[wid, j], sv)
        pltpu.async_copy(h_sh.at[gv.at[0]], rows_v, sem).wait()
        pltpu.sync_copy(rows_v, acc_sh.at[sv.at[0]], add=True)
        return carry

    lax.fori_loop(0, nch, body, 0)
    plsc.subcore_barrier()

    @pl.when(s == 0)
    def _():
        pltpu.sync_copy(acc_sh, out_hbm.at[c])


def _make_deg(n, e):
    nch = e // (NW * CHUNK)
    return functools.partial(
        pl.kernel,
        mesh=_mesh(),
        out_type=jax.ShapeDtypeStruct((NC, n), jnp.float32),
        scratch_types=[
            pltpu.VMEM((nch, CHUNK), jnp.int32),
            pltpu.VMEM((CHUNK,), jnp.float32),
            pltpu.VMEM_SHARED((n,), jnp.float32),
            pltpu.SemaphoreType.DMA,
        ],
    )(functools.partial(_deg_body, n, nch))


def _make_agg(n, d, e):
    nch = e // (NW * CHUNK)
    opc = CHUNK * d
    return functools.partial(
        pl.kernel,
        mesh=_mesh(),
        out_type=jax.ShapeDtypeStruct((NC, n * d), jnp.float32),
        scratch_types=[
            pltpu.VMEM((1, opc), jnp.int32),
            pltpu.VMEM((1, opc), jnp.int32),
            pltpu.VMEM((opc,), jnp.float32),
            pltpu.VMEM_SHARED((n * d,), jnp.float32),
            pltpu.VMEM_SHARED((n * d,), jnp.float32),
            pltpu.SemaphoreType.DMA,
        ],
    )(functools.partial(_agg_body, nch))


def _mm_body(x_ref, w_ref, o_ref):
    o_ref[...] = jnp.dot(x_ref[...], w_ref[...],
                         preferred_element_type=jnp.float32)


def _scale_body(indeg_ref, g_ref, dis_ref, hp_ref):
    deg = indeg_ref[0] + indeg_ref[1] + 1.0
    dis = lax.rsqrt(deg)
    dis_ref[...] = dis
    hp_ref[...] = dis[:, None] * g_ref[...]


def _mid_body(dis_ref, agg_ref, hp_ref, b1_ref, w2_ref, h2p_ref):
    dis = dis_ref[...]
    a = agg_ref[0] + agg_ref[1] + hp_ref[...]
    h1 = jnp.maximum(dis[:, None] * a + b1_ref[...][None, :], 0.0)
    g2 = jnp.dot(h1, w2_ref[...], preferred_element_type=jnp.float32)
    h2p_ref[...] = dis[:, None] * g2


def _out_body(dis_ref, agg_ref, h2p_ref, b2_ref, o_ref):
    dis = dis_ref[...]
    a = agg_ref[0] + agg_ref[1] + h2p_ref[...]
    o_ref[...] = dis[:, None] * a + b2_ref[...][None, :]


def _offsets(nodes, d, nch):
    flat = (nodes[:, None] * d
            + jnp.arange(d, dtype=jnp.int32)[None, :]).reshape(-1)
    return flat.reshape(NW, nch, 1, CHUNK * d)


def kernel(x, edge_index, W1, b1, W2, b2):
    n, _ = x.shape
    e = edge_index.shape[1]
    hid = W1.shape[1]
    out_dim = W2.shape[1]
    nch = e // (NW * CHUNK)
    assert e == NW * nch * CHUNK

    ei = edge_index.astype(jnp.int32)
    src, dst = ei[0], ei[1]
    dst3 = dst.reshape(NW, nch, CHUNK)
    g1offs = _offsets(src, hid, nch)
    s1offs = _offsets(dst, hid, nch)
    g2offs = _offsets(src, out_dim, nch)
    s2offs = _offsets(dst, out_dim, nch)
    zeros1 = jnp.zeros((n,), jnp.float32)
    zeros_h = jnp.zeros((n * hid,), jnp.float32)
    zeros_o = jnp.zeros((n * out_dim,), jnp.float32)

    g1 = pl.pallas_call(
        _mm_body,
        out_shape=jax.ShapeDtypeStruct((n, hid), jnp.float32),
    )(x, W1)

    indeg = _make_deg(n, e)(dst3, zeros1)

    dis, h1p = pl.pallas_call(
        _scale_body,
        out_shape=(jax.ShapeDtypeStruct((n,), jnp.float32),
                   jax.ShapeDtypeStruct((n, hid), jnp.float32)),
    )(indeg, g1)

    agg1 = _make_agg(n, hid, e)(
        g1offs, s1offs, h1p.reshape(-1), zeros_h).reshape(NC, n, hid)

    h2p = pl.pallas_call(
        _mid_body,
        out_shape=jax.ShapeDtypeStruct((n, out_dim), jnp.float32),
    )(dis, agg1, h1p, b1, W2)

    agg2 = _make_agg(n, out_dim, e)(
        g2offs, s2offs, h2p.reshape(-1), zeros_o).reshape(NC, n, out_dim)

    out = pl.pallas_call(
        _out_body,
        out_shape=jax.ShapeDtypeStruct((n, out_dim), jnp.float32),
    )(dis, agg2, h2p, b2)

    return out
